# baseline (speedup 1.0000x reference)
import functools

import jax
import jax.numpy as jnp
from jax import lax
from jax.experimental import pallas as pl
from jax.experimental.pallas import tpu as pltpu
from jax.experimental.pallas import tpu_sc as plsc

VOCAB = 10000
EMBED_DIM = 64
BATCH = 16384
BAG_LEN = 50

NUM_CORES = 2
NUM_SUBCORES = 16
LANES = 16
NUM_WORKERS = NUM_CORES * NUM_SUBCORES
BAGS_PER_W = BATCH // NUM_WORKERS
IDX_PER_W = BAGS_PER_W * BAG_LEN
GROUPS = BAGS_PER_W // LANES


FOLD_BLK = 1024
VOCAB_PAD = 10240


def _fold_body(table_ref, w1_ref, b1_ref, w2p_ref, b2_ref, s_hbm, s_scr, sem):
    i = pl.program_id(0)
    vvp = lax.dot_general(
        w2p_ref[...], w1_ref[...], (((1,), (0,)), ((), ())),
        preferred_element_type=jnp.float32,
    )
    c = jnp.sum(b1_ref[...] * w2p_ref[0:1, :]) + b2_ref[0, 0]
    s_blk = lax.dot_general(
        vvp, table_ref[...], (((1,), (1,)), ((), ())),
        preferred_element_type=jnp.float32,
    )
    s_scr[...] = (s_blk[0:1, :] + c) * (1.0 / BAG_LEN)
    cp = pltpu.make_async_copy(
        s_scr, s_hbm.at[0:1, pl.ds(i * FOLD_BLK, FOLD_BLK)], sem
    )
    cp.start()
    cp.wait()


def _fold_tables(emb_table, W1, b1, W2, b2):
    w2p = jnp.pad(W2, ((0, 7), (0, 0)))
    return pl.pallas_call(
        _fold_body,
        grid=(VOCAB_PAD // FOLD_BLK,),
        in_specs=[
            pl.BlockSpec((FOLD_BLK, EMBED_DIM), lambda i: (i, 0)),
            pl.BlockSpec((EMBED_DIM, EMBED_DIM), lambda i: (0, 0)),
            pl.BlockSpec((1, EMBED_DIM), lambda i: (0, 0)),
            pl.BlockSpec((8, EMBED_DIM), lambda i: (0, 0)),
            pl.BlockSpec((1, 1), lambda i: (0, 0)),
        ],
        out_specs=pl.BlockSpec(memory_space=pltpu.HBM),
        out_shape=jax.ShapeDtypeStruct((1, VOCAB_PAD), jnp.float32),
        scratch_shapes=[
            pltpu.VMEM((1, FOLD_BLK), jnp.float32),
            pltpu.SemaphoreType.DMA,
        ],
    )(emb_table, W1, b1.reshape(1, EMBED_DIM), w2p, b2.reshape(1, 1))


@functools.partial(
    pl.kernel,
    mesh=plsc.VectorSubcoreMesh(core_axis_name="c", subcore_axis_name="s"),
    out_type=jax.ShapeDtypeStruct((BATCH,), jnp.float32),
    compiler_params=pltpu.CompilerParams(needs_layout_passes=False),
    scratch_types=[
        pltpu.VMEM((VOCAB_PAD,), jnp.float32),
        pltpu.VMEM((BAG_LEN, BAGS_PER_W), jnp.int32),
        pltpu.VMEM((BAGS_PER_W,), jnp.float32),
        pltpu.SemaphoreType.DMA,
        pltpu.SemaphoreType.DMA,
    ],
)
def _sc_bag_sum(s_hbm, idxt_hbm, out_hbm, s_v, idx_v, out_v, sem_s, sem_i):
    wid = lax.axis_index("s") * NUM_CORES + lax.axis_index("c")
    bag0 = wid * BAGS_PER_W
    cp_s = pltpu.async_copy(s_hbm.at[0], s_v, sem_s)
    cp_i = pltpu.async_copy(idxt_hbm.at[:, pl.ds(bag0, BAGS_PER_W)], idx_v, sem_i)
    cp_s.wait()
    cp_i.wait()

    def group_body(j, carry):
        col = j * LANES
        accs = [jnp.zeros((16,), jnp.float32) for _ in range(4)]
        for l in range(BAG_LEN):
            iv = idx_v[l, pl.ds(col, LANES)]
            accs[l % 4] = accs[l % 4] + plsc.load_gather(s_v, [iv])
        out_v[pl.ds(col, LANES)] = (accs[0] + accs[1]) + (accs[2] + accs[3])
        return carry

    lax.fori_loop(0, GROUPS, group_body, 0)
    pltpu.sync_copy(out_v, out_hbm.at[pl.ds(bag0, BAGS_PER_W)])


def kernel(input, emb_table, W1, b1, W2, b2):
    s = _fold_tables(emb_table, W1, b1, W2, b2)
    idx_t = input.astype(jnp.int32).T
    out = _sc_bag_sum(s, idx_t)
    return out.reshape(BATCH, 1)


# speedup vs baseline: 1.1814x; 1.1814x over previous
import functools

import jax
import jax.numpy as jnp
from jax import lax
from jax.experimental import pallas as pl
from jax.experimental.pallas import tpu as pltpu
from jax.experimental.pallas import tpu_sc as plsc

VOCAB = 10000
EMBED_DIM = 64
BATCH = 16384
BAG_LEN = 50

NUM_CORES = 2
NUM_SUBCORES = 16
LANES = 16
NUM_WORKERS = NUM_CORES * NUM_SUBCORES
BAGS_PER_W = BATCH // NUM_WORKERS
IDX_PER_W = BAGS_PER_W * BAG_LEN
GROUPS = BAGS_PER_W // LANES


def _fold_body(table_hbm, w1_ref, b1_ref, w2p_ref, b2_ref, s_ref, table_scr, sem):
    cp = pltpu.make_async_copy(table_hbm, table_scr, sem)
    cp.start()
    vvp = lax.dot_general(
        w2p_ref[...], w1_ref[...], (((1,), (0,)), ((), ())),
        preferred_element_type=jnp.float32,
    )
    c = jnp.sum(b1_ref[...] * w2p_ref[0:1, :]) + b2_ref[0, 0]
    cp.wait()
    s_blk = lax.dot_general(
        vvp, table_scr[...], (((1,), (1,)), ((), ())),
        preferred_element_type=jnp.float32,
    )
    s_ref[...] = (s_blk[0:1, :] + c) * (1.0 / BAG_LEN)


def _fold_tables(emb_table, W1, b1, W2, b2):
    w2p = jnp.pad(W2, ((0, 7), (0, 0)))
    return pl.pallas_call(
        _fold_body,
        in_specs=[
            pl.BlockSpec(memory_space=pltpu.HBM),
            pl.BlockSpec((EMBED_DIM, EMBED_DIM), lambda: (0, 0)),
            pl.BlockSpec((1, EMBED_DIM), lambda: (0, 0)),
            pl.BlockSpec((8, EMBED_DIM), lambda: (0, 0)),
            pl.BlockSpec((1, 1), lambda: (0, 0)),
        ],
        out_specs=pl.BlockSpec((1, VOCAB), lambda: (0, 0)),
        out_shape=jax.ShapeDtypeStruct((1, VOCAB), jnp.float32),
        scratch_shapes=[
            pltpu.VMEM((VOCAB, EMBED_DIM), jnp.float32),
            pltpu.SemaphoreType.DMA,
        ],
    )(emb_table, W1, b1.reshape(1, EMBED_DIM), w2p, b2.reshape(1, 1))


@functools.partial(
    pl.kernel,
    mesh=plsc.VectorSubcoreMesh(core_axis_name="c", subcore_axis_name="s"),
    out_type=jax.ShapeDtypeStruct((BATCH,), jnp.float32),
    compiler_params=pltpu.CompilerParams(needs_layout_passes=False),
    scratch_types=[
        pltpu.VMEM((VOCAB,), jnp.float32),
        pltpu.VMEM((BAG_LEN, BAGS_PER_W), jnp.int32),
        pltpu.VMEM((BAGS_PER_W,), jnp.float32),
        pltpu.SemaphoreType.DMA,
        pltpu.SemaphoreType.DMA,
    ],
)
def _sc_bag_sum(s_hbm, idxt_hbm, out_hbm, s_v, idx_v, out_v, sem_s, sem_i):
    wid = lax.axis_index("s") * NUM_CORES + lax.axis_index("c")
    bag0 = wid * BAGS_PER_W
    cp_s = pltpu.async_copy(s_hbm.at[0], s_v, sem_s)
    cp_i = pltpu.async_copy(idxt_hbm.at[:, pl.ds(bag0, BAGS_PER_W)], idx_v, sem_i)
    cp_s.wait()
    cp_i.wait()

    @plsc.parallel_loop(0, GROUPS, 1)
    def group_body(j):
        col = j * LANES
        accs = [jnp.zeros((16,), jnp.float32) for _ in range(4)]
        for l in range(BAG_LEN):
            iv = idx_v[l, pl.ds(col, LANES)]
            accs[l % 4] = accs[l % 4] + plsc.load_gather(s_v, [iv])
        out_v[pl.ds(col, LANES)] = (accs[0] + accs[1]) + (accs[2] + accs[3])
    pltpu.sync_copy(out_v, out_hbm.at[pl.ds(bag0, BAGS_PER_W)])


def kernel(input, emb_table, W1, b1, W2, b2):
    s = _fold_tables(emb_table, W1, b1, W2, b2)
    idx_t = input.astype(jnp.int32).T
    out = _sc_bag_sum(s, idx_t)
    return out.reshape(BATCH, 1)


# speedup vs baseline: 1.2074x; 1.0220x over previous
import functools

import jax
import jax.numpy as jnp
from jax import lax
from jax.experimental import pallas as pl
from jax.experimental.pallas import tpu as pltpu
from jax.experimental.pallas import tpu_sc as plsc

VOCAB = 10000
EMBED_DIM = 64
BATCH = 16384
BAG_LEN = 50

NUM_CORES = 2
NUM_SUBCORES = 16
LANES = 16
NUM_WORKERS = NUM_CORES * NUM_SUBCORES
BAGS_PER_W = BATCH // NUM_WORKERS
IDX_PER_W = BAGS_PER_W * BAG_LEN
GROUPS = BAGS_PER_W // LANES


def _fold_body(table_ref, w1_ref, b1_ref, w2p_ref, b2_ref, s_ref):
    vvp = lax.dot_general(
        w2p_ref[...], w1_ref[...], (((1,), (0,)), ((), ())),
        preferred_element_type=jnp.float32,
    )
    c = jnp.sum(b1_ref[...] * w2p_ref[0:1, :]) + b2_ref[0, 0]
    s_blk = lax.dot_general(
        vvp, table_ref[...], (((1,), (1,)), ((), ())),
        preferred_element_type=jnp.float32,
    )
    s_ref[...] = (s_blk[0:1, :] + c) * (1.0 / BAG_LEN)


def _fold_tables(emb_table, W1, b1, W2, b2):
    w2p = jnp.pad(W2, ((0, 7), (0, 0)))
    return pl.pallas_call(
        _fold_body,
        out_shape=jax.ShapeDtypeStruct((1, VOCAB), jnp.float32),
    )(emb_table, W1, b1.reshape(1, EMBED_DIM), w2p, b2.reshape(1, 1))


@functools.partial(
    pl.kernel,
    mesh=plsc.VectorSubcoreMesh(core_axis_name="c", subcore_axis_name="s"),
    out_type=jax.ShapeDtypeStruct((BATCH,), jnp.float32),
    compiler_params=pltpu.CompilerParams(needs_layout_passes=False),
    scratch_types=[
        pltpu.VMEM((VOCAB,), jnp.float32),
        pltpu.VMEM((BAG_LEN, BAGS_PER_W), jnp.int32),
        pltpu.VMEM((BAGS_PER_W,), jnp.float32),
        pltpu.SemaphoreType.DMA,
        pltpu.SemaphoreType.DMA,
    ],
)
def _sc_bag_sum(s_hbm, idxt_hbm, out_hbm, s_v, idx_v, out_v, sem_s, sem_i):
    wid = lax.axis_index("s") * NUM_CORES + lax.axis_index("c")
    bag0 = wid * BAGS_PER_W
    cp_s = pltpu.async_copy(s_hbm.at[0], s_v, sem_s)
    cp_i = pltpu.async_copy(idxt_hbm.at[:, pl.ds(bag0, BAGS_PER_W)], idx_v, sem_i)
    cp_s.wait()
    cp_i.wait()

    @plsc.parallel_loop(0, GROUPS, 1)
    def group_body(j):
        col = j * LANES
        accs = [jnp.zeros((16,), jnp.float32) for _ in range(4)]
        for l in range(BAG_LEN):
            iv = idx_v[l, pl.ds(col, LANES)]
            accs[l % 4] = accs[l % 4] + plsc.load_gather(s_v, [iv])
        out_v[pl.ds(col, LANES)] = (accs[0] + accs[1]) + (accs[2] + accs[3])
    pltpu.sync_copy(out_v, out_hbm.at[pl.ds(bag0, BAGS_PER_W)])


def kernel(input, emb_table, W1, b1, W2, b2):
    s = _fold_tables(emb_table, W1, b1, W2, b2)
    idx_t = input.astype(jnp.int32).T
    out = _sc_bag_sum(s, idx_t)
    return out.reshape(BATCH, 1)
